# baseline (speedup 1.0000x reference)
import functools

import jax
import jax.numpy as jnp
import numpy as np
from jax.experimental import pallas as pl
from jax.experimental.pallas import tpu as pltpu

B, S, H = 2, 2048, 1024
NHS, HD = 4, 64
HS = NHS * HD
E, K = 8, 2
INTER = 768
EPS = 1e-05

CHUNK = 256
NCHUNK = S // CHUNK


def _rms(x):
    var = jnp.mean(x * x, axis=-1, keepdims=True)
    return x * jax.lax.rsqrt(var + EPS)


def _add_kernel(a_ref, b_ref, o_ref):
    o_ref[...] = a_ref[...] + b_ref[...]


def _make_hs(hidden, inj):
    blk = pl.BlockSpec((1, 512, H), lambda b, c: (b, c, 0))
    return pl.pallas_call(
        _add_kernel,
        grid=(B, S // 512),
        in_specs=[blk, blk],
        out_specs=blk,
        out_shape=jax.ShapeDtypeStruct((B, S, H), jnp.float32),
    )(hidden, inj)


def _router_kernel(hs0_ref, wg_ref, topi_ref, wv_ref, aux_ref):
    x = hs0_ref[...]
    logits = jax.lax.dot_general(
        x, wg_ref[...], (((1,), (0,)), ((), ())),
        preferred_element_type=jnp.float32)
    m = jnp.max(logits, axis=1, keepdims=True)
    ex = jnp.exp(logits - m)
    p = ex / jnp.sum(ex, axis=1, keepdims=True)

    iota = jax.lax.broadcasted_iota(jnp.int32, (B, E), 1)
    v1 = jnp.max(p, axis=1, keepdims=True)
    i1 = jnp.min(jnp.where(p == v1, iota, E), axis=1, keepdims=True)
    mask1 = iota == i1
    p2 = jnp.where(mask1, -1.0, p)
    v2 = jnp.max(p2, axis=1, keepdims=True)
    i2 = jnp.min(jnp.where(p2 == v2, iota, E), axis=1, keepdims=True)
    mask2 = iota == i2

    denom = jnp.maximum(v1 + v2, 1e-08)
    wv_ref[...] = jnp.concatenate([v1 / denom, v2 / denom], axis=1)
    topi_ref[...] = jnp.concatenate([i1, i2], axis=1)

    importance = jnp.sum(p, axis=0, keepdims=True) / B
    sel = (mask1 | mask2).astype(jnp.float32)
    load = jnp.sum(sel, axis=0, keepdims=True) / (B * K)
    aux_ref[...] = jnp.sum(E * importance * load, axis=1, keepdims=True)


def _route(hs0, wg):
    return pl.pallas_call(
        _router_kernel,
        in_specs=[pl.BlockSpec(memory_space=pltpu.VMEM),
                  pl.BlockSpec(memory_space=pltpu.VMEM)],
        out_specs=[pl.BlockSpec(memory_space=pltpu.VMEM)] * 3,
        out_shape=[
            jax.ShapeDtypeStruct((B, K), jnp.int32),
            jax.ShapeDtypeStruct((B, K), jnp.float32),
            jax.ShapeDtypeStruct((1, 1), jnp.float32),
        ],
    )(hs0, wg)


def _moe_kernel(topi_ref, wv_ref,
                hs_ref, cos_ref, sin_ref,
                wd_ref, wqkv_ref, wo_ref, wgu_ref, wdn_ref, wu_ref,
                out_ref,
                h_s, q_s, k_s, v_s):
    b = pl.program_id(0)
    kk = pl.program_id(1)
    wgt = wv_ref[b, kk]

    bf = jnp.bfloat16
    wd = wd_ref[0].astype(bf)
    wqkv = wqkv_ref[0].astype(bf)

    def rope(x, cosc, sinc):
        outs = []
        for hh in range(NHS):
            xh = x[:, hh * HD:(hh + 1) * HD]
            x1 = xh[:, :HD // 2]
            x2 = xh[:, HD // 2:]
            rot = jnp.concatenate([-x2, x1], axis=1)
            outs.append(xh * cosc + rot * sinc)
        return jnp.concatenate(outs, axis=1)

    def phase1(c, carry):
        rows = pl.ds(c * CHUNK, CHUNK)
        x = hs_ref[0, rows, :].astype(bf)
        hc = jnp.dot(x, wd, preferred_element_type=jnp.float32)
        h_s[rows, :] = hc
        qkv = jnp.dot(hc.astype(bf), wqkv,
                      preferred_element_type=jnp.float32)
        cosc = cos_ref[rows, :]
        sinc = sin_ref[rows, :]
        q_s[rows, :] = rope(qkv[:, :HS], cosc, sinc).astype(bf)
        k_s[rows, :] = rope(qkv[:, HS:2 * HS], cosc, sinc).astype(bf)
        v_s[rows, :] = qkv[:, 2 * HS:].astype(bf)
        return carry

    jax.lax.fori_loop(0, NCHUNK, phase1, 0)

    wo = wo_ref[0].astype(bf)
    wgu = wgu_ref[0].astype(bf)
    wdn = wdn_ref[0].astype(bf)
    wu = wu_ref[0].astype(bf)
    scale = 1.0 / np.sqrt(HD).astype(np.float32)

    def phase2(c, carry):
        rows = pl.ds(c * CHUNK, CHUNK)
        hc = h_s[rows, :]
        o_heads = []
        for hh in range(NHS):
            cols = slice(hh * HD, (hh + 1) * HD)
            qh = q_s[rows, cols]
            kh = k_s[:, cols]
            vh = v_s[:, cols]
            scores = jax.lax.dot_general(
                qh, kh, (((1,), (1,)), ((), ())),
                preferred_element_type=jnp.float32) * scale
            mx = jnp.max(scores, axis=1, keepdims=True)
            ee = jnp.exp(scores - mx)
            attn = (ee / jnp.sum(ee, axis=1, keepdims=True)).astype(bf)
            o_heads.append(jnp.dot(attn, vh, preferred_element_type=jnp.float32))
        o = jnp.concatenate(o_heads, axis=1).astype(bf)
        o = jnp.dot(o, wo, preferred_element_type=jnp.float32)
        t = _rms(hc + o)
        gu = jnp.dot(t.astype(bf), wgu,
                     preferred_element_type=jnp.float32)
        g = gu[:, :INTER]
        u = gu[:, INTER:]
        mm = jnp.dot((jax.nn.silu(g) * u).astype(bf), wdn,
                     preferred_element_type=jnp.float32)
        t2 = _rms(t + mm)
        oe = jnp.dot(t2.astype(bf), wu, preferred_element_type=jnp.float32)

        @pl.when(kk == 0)
        def _():
            out_ref[0, rows, :] = wgt * oe

        @pl.when(kk != 0)
        def _():
            out_ref[0, rows, :] = out_ref[0, rows, :] + wgt * oe

        return carry

    jax.lax.fori_loop(0, NCHUNK, phase2, 0)


def _moe(topi, wv, hs, cos, sin, Wd, Wqkv, Wo, Wgu, Wdn, Wu):
    grid_spec = pltpu.PrefetchScalarGridSpec(
        num_scalar_prefetch=2,
        grid=(B, K),
        in_specs=[
            pl.BlockSpec((1, S, H), lambda b, k, ti, wv_: (b, 0, 0)),
            pl.BlockSpec((S, HD), lambda b, k, ti, wv_: (0, 0)),
            pl.BlockSpec((S, HD), lambda b, k, ti, wv_: (0, 0)),
            pl.BlockSpec((1, H, HS), lambda b, k, ti, wv_: (ti[b, k], 0, 0)),
            pl.BlockSpec((1, HS, 3 * HS), lambda b, k, ti, wv_: (ti[b, k], 0, 0)),
            pl.BlockSpec((1, HS, HS), lambda b, k, ti, wv_: (ti[b, k], 0, 0)),
            pl.BlockSpec((1, HS, 2 * INTER), lambda b, k, ti, wv_: (ti[b, k], 0, 0)),
            pl.BlockSpec((1, INTER, HS), lambda b, k, ti, wv_: (ti[b, k], 0, 0)),
            pl.BlockSpec((1, HS, H), lambda b, k, ti, wv_: (ti[b, k], 0, 0)),
        ],
        out_specs=pl.BlockSpec((1, S, H), lambda b, k, ti, wv_: (b, 0, 0)),
        scratch_shapes=[
            pltpu.VMEM((S, HS), jnp.float32),
            pltpu.VMEM((S, HS), jnp.bfloat16),
            pltpu.VMEM((S, HS), jnp.bfloat16),
            pltpu.VMEM((S, HS), jnp.bfloat16),
        ],
    )
    return pl.pallas_call(
        _moe_kernel,
        grid_spec=grid_spec,
        out_shape=jax.ShapeDtypeStruct((B, S, H), jnp.float32),
        compiler_params=pltpu.CompilerParams(
            vmem_limit_bytes=64 * 1024 * 1024,
            dimension_semantics=("parallel", "arbitrary"),
        ),
    )(topi, wv, hs, cos, sin, Wd, Wqkv, Wo, Wgu, Wdn, Wu)


def _final_kernel(hs_ref, mix_ref, o_ref):
    x = hs_ref[...] + mix_ref[...]
    var = jnp.mean(x * x, axis=-1, keepdims=True)
    o_ref[...] = x * jax.lax.rsqrt(var + EPS)


def _finalize(hs, mixed):
    blk = pl.BlockSpec((1, 512, H), lambda b, c: (b, c, 0))
    return pl.pallas_call(
        _final_kernel,
        grid=(B, S // 512),
        in_specs=[blk, blk],
        out_specs=blk,
        out_shape=jax.ShapeDtypeStruct((B, S, H), jnp.float32),
    )(hs, mixed)


def kernel(hidden_states, input_injection, cos, sin, Wg, Wd, Wu, Wqkv, Wo, Wgu, Wdn):
    hs = _make_hs(hidden_states, input_injection)
    topi, wv, aux = _route(hs[:, 0, :], Wg)
    mixed = _moe(topi, wv, hs, cos, sin, Wd, Wqkv, Wo, Wgu, Wdn, Wu)
    out = _finalize(hs, mixed)
    return out, aux.reshape(())


# speedup vs baseline: 1.0914x; 1.0914x over previous
import functools

import jax
import jax.numpy as jnp
import numpy as np
from jax.experimental import pallas as pl
from jax.experimental.pallas import tpu as pltpu

B, S, H = 2, 2048, 1024
NHS, HD = 4, 64
HS = NHS * HD
E, K = 8, 2
INTER = 768
EPS = 1e-05

CHUNK = 256
NCHUNK = S // CHUNK


def _rms(x):
    var = jnp.mean(x * x, axis=-1, keepdims=True)
    return x * jax.lax.rsqrt(var + EPS)


def _add_kernel(a_ref, b_ref, o_ref):
    o_ref[...] = a_ref[...] + b_ref[...]


def _make_hs(hidden, inj):
    blk = pl.BlockSpec((1, 512, H), lambda b, c: (b, c, 0))
    return pl.pallas_call(
        _add_kernel,
        grid=(B, S // 512),
        in_specs=[blk, blk],
        out_specs=blk,
        out_shape=jax.ShapeDtypeStruct((B, S, H), jnp.float32),
    )(hidden, inj)


def _router_kernel(hs0_ref, wg_ref, topi_ref, wv_ref, aux_ref):
    x = hs0_ref[...]
    logits = jax.lax.dot_general(
        x, wg_ref[...], (((1,), (0,)), ((), ())),
        preferred_element_type=jnp.float32)
    m = jnp.max(logits, axis=1, keepdims=True)
    ex = jnp.exp(logits - m)
    p = ex / jnp.sum(ex, axis=1, keepdims=True)

    iota = jax.lax.broadcasted_iota(jnp.int32, (B, E), 1)
    v1 = jnp.max(p, axis=1, keepdims=True)
    i1 = jnp.min(jnp.where(p == v1, iota, E), axis=1, keepdims=True)
    mask1 = iota == i1
    p2 = jnp.where(mask1, -1.0, p)
    v2 = jnp.max(p2, axis=1, keepdims=True)
    i2 = jnp.min(jnp.where(p2 == v2, iota, E), axis=1, keepdims=True)
    mask2 = iota == i2

    denom = jnp.maximum(v1 + v2, 1e-08)
    wv_ref[...] = jnp.concatenate([v1 / denom, v2 / denom], axis=1)
    topi_ref[...] = jnp.concatenate([i1, i2], axis=1)

    importance = jnp.sum(p, axis=0, keepdims=True) / B
    sel = (mask1 | mask2).astype(jnp.float32)
    load = jnp.sum(sel, axis=0, keepdims=True) / (B * K)
    aux_ref[...] = jnp.sum(E * importance * load, axis=1, keepdims=True)


def _route(hs0, wg):
    return pl.pallas_call(
        _router_kernel,
        in_specs=[pl.BlockSpec(memory_space=pltpu.VMEM),
                  pl.BlockSpec(memory_space=pltpu.VMEM)],
        out_specs=[pl.BlockSpec(memory_space=pltpu.VMEM)] * 3,
        out_shape=[
            jax.ShapeDtypeStruct((B, K), jnp.int32),
            jax.ShapeDtypeStruct((B, K), jnp.float32),
            jax.ShapeDtypeStruct((1, 1), jnp.float32),
        ],
    )(hs0, wg)


def _moe_kernel(topi_ref, wv_ref,
                hs_ref, cos_ref, sin_ref,
                wd_ref, wqkv_ref, wo_ref, wgu_ref, wdn_ref, wu_ref,
                out_ref,
                h_s, q_s, k_s, v_s):
    b = pl.program_id(0)
    kk = pl.program_id(1)
    wgt = wv_ref[b, kk]

    bf = jnp.bfloat16
    wd = wd_ref[0].astype(bf)
    wqkv = wqkv_ref[0].astype(bf)
    scale = np.float32(1.0 / np.sqrt(HD))

    def rope_head(xh, cosc, sinc):
        x1 = xh[:, :HD // 2]
        x2 = xh[:, HD // 2:]
        rot = jnp.concatenate([-x2, x1], axis=1)
        return xh * cosc + rot * sinc

    def phase1(c, carry):
        rows = pl.ds(c * CHUNK, CHUNK)
        x = hs_ref[0, rows, :].astype(bf)
        hc = jnp.dot(x, wd, preferred_element_type=jnp.float32)
        h_s[rows, :] = hc
        qkv = jnp.dot(hc.astype(bf), wqkv,
                      preferred_element_type=jnp.float32)
        cosc = cos_ref[rows, :]
        sinc = sin_ref[rows, :]
        for hh in range(NHS):
            qh = qkv[:, hh * HD:(hh + 1) * HD]
            kh = qkv[:, HS + hh * HD:HS + (hh + 1) * HD]
            vh = qkv[:, 2 * HS + hh * HD:2 * HS + (hh + 1) * HD]
            q_s[hh, rows, :] = (rope_head(qh, cosc, sinc) * scale).astype(bf)
            k_s[hh, rows, :] = rope_head(kh, cosc, sinc).astype(bf)
            v_s[hh, rows, :] = vh.astype(bf)
        return carry

    jax.lax.fori_loop(0, NCHUNK, phase1, 0)

    wo = wo_ref[0].astype(bf)
    wgu = wgu_ref[0].astype(bf)
    wdn = wdn_ref[0].astype(bf)
    wu = wu_ref[0].astype(bf)

    def phase2(c, carry):
        rows = pl.ds(c * CHUNK, CHUNK)
        hc = h_s[rows, :]
        o_heads = []
        for hh in range(NHS):
            qh = q_s[hh, rows, :]
            kh = k_s[hh]
            vh = v_s[hh]
            scores = jax.lax.dot_general(
                qh, kh, (((1,), (1,)), ((), ())),
                preferred_element_type=jnp.float32)
            mx = jnp.max(scores, axis=1, keepdims=True)
            ee = jnp.exp(scores - mx)
            rs = 1.0 / jnp.sum(ee, axis=1, keepdims=True)
            ov = jnp.dot(ee.astype(bf), vh,
                         preferred_element_type=jnp.float32)
            o_heads.append(ov * rs)
        o = jnp.concatenate(o_heads, axis=1).astype(bf)
        o = jnp.dot(o, wo, preferred_element_type=jnp.float32)
        t = _rms(hc + o)
        gu = jnp.dot(t.astype(bf), wgu,
                     preferred_element_type=jnp.float32)
        g = gu[:, :INTER]
        u = gu[:, INTER:]
        mm = jnp.dot((jax.nn.silu(g) * u).astype(bf), wdn,
                     preferred_element_type=jnp.float32)
        t2 = _rms(t + mm)
        oe = jnp.dot(t2.astype(bf), wu, preferred_element_type=jnp.float32)

        @pl.when(kk == 0)
        def _():
            out_ref[0, rows, :] = wgt * oe

        @pl.when(kk != 0)
        def _():
            out_ref[0, rows, :] = out_ref[0, rows, :] + wgt * oe

        return carry

    jax.lax.fori_loop(0, NCHUNK, phase2, 0)


def _moe(topi, wv, hs, cos, sin, Wd, Wqkv, Wo, Wgu, Wdn, Wu):
    grid_spec = pltpu.PrefetchScalarGridSpec(
        num_scalar_prefetch=2,
        grid=(B, K),
        in_specs=[
            pl.BlockSpec((1, S, H), lambda b, k, ti, wv_: (b, 0, 0)),
            pl.BlockSpec((S, HD), lambda b, k, ti, wv_: (0, 0)),
            pl.BlockSpec((S, HD), lambda b, k, ti, wv_: (0, 0)),
            pl.BlockSpec((1, H, HS), lambda b, k, ti, wv_: (ti[b, k], 0, 0)),
            pl.BlockSpec((1, HS, 3 * HS), lambda b, k, ti, wv_: (ti[b, k], 0, 0)),
            pl.BlockSpec((1, HS, HS), lambda b, k, ti, wv_: (ti[b, k], 0, 0)),
            pl.BlockSpec((1, HS, 2 * INTER), lambda b, k, ti, wv_: (ti[b, k], 0, 0)),
            pl.BlockSpec((1, INTER, HS), lambda b, k, ti, wv_: (ti[b, k], 0, 0)),
            pl.BlockSpec((1, HS, H), lambda b, k, ti, wv_: (ti[b, k], 0, 0)),
        ],
        out_specs=pl.BlockSpec((1, S, H), lambda b, k, ti, wv_: (b, 0, 0)),
        scratch_shapes=[
            pltpu.VMEM((S, HS), jnp.float32),
            pltpu.VMEM((NHS, S, HD), jnp.bfloat16),
            pltpu.VMEM((NHS, S, HD), jnp.bfloat16),
            pltpu.VMEM((NHS, S, HD), jnp.bfloat16),
        ],
    )
    return pl.pallas_call(
        _moe_kernel,
        grid_spec=grid_spec,
        out_shape=jax.ShapeDtypeStruct((B, S, H), jnp.float32),
        compiler_params=pltpu.CompilerParams(
            vmem_limit_bytes=64 * 1024 * 1024,
            dimension_semantics=("parallel", "arbitrary"),
        ),
    )(topi, wv, hs, cos, sin, Wd, Wqkv, Wo, Wgu, Wdn, Wu)


def _final_kernel(hs_ref, mix_ref, o_ref):
    x = hs_ref[...] + mix_ref[...]
    var = jnp.mean(x * x, axis=-1, keepdims=True)
    o_ref[...] = x * jax.lax.rsqrt(var + EPS)


def _finalize(hs, mixed):
    blk = pl.BlockSpec((1, 512, H), lambda b, c: (b, c, 0))
    return pl.pallas_call(
        _final_kernel,
        grid=(B, S // 512),
        in_specs=[blk, blk],
        out_specs=blk,
        out_shape=jax.ShapeDtypeStruct((B, S, H), jnp.float32),
    )(hs, mixed)


def kernel(hidden_states, input_injection, cos, sin, Wg, Wd, Wu, Wqkv, Wo, Wgu, Wdn):
    hs = _make_hs(hidden_states, input_injection)
    topi, wv, aux = _route(hs[:, 0, :], Wg)
    mixed = _moe(topi, wv, hs, cos, sin, Wd, Wqkv, Wo, Wgu, Wdn, Wu)
    out = _finalize(hs, mixed)
    return out, aux.reshape(())


# speedup vs baseline: 1.1591x; 1.0621x over previous
import functools

import jax
import jax.numpy as jnp
import numpy as np
from jax.experimental import pallas as pl
from jax.experimental.pallas import tpu as pltpu

B, S, H = 2, 2048, 1024
NHS, HD = 4, 64
HS = NHS * HD
E, K = 8, 2
INTER = 768
EPS = 1e-05

CHUNK = 256
NCHUNK = S // CHUNK


def _rms(x):
    var = jnp.mean(x * x, axis=-1, keepdims=True)
    return x * jax.lax.rsqrt(var + EPS)


def _add_kernel(a_ref, b_ref, o_ref):
    o_ref[...] = a_ref[...] + b_ref[...]


def _make_hs(hidden, inj):
    blk = pl.BlockSpec((1, 512, H), lambda b, c: (b, c, 0))
    return pl.pallas_call(
        _add_kernel,
        grid=(B, S // 512),
        in_specs=[blk, blk],
        out_specs=blk,
        out_shape=jax.ShapeDtypeStruct((B, S, H), jnp.float32),
    )(hidden, inj)


def _router_kernel(hs0_ref, wg_ref, topi_ref, wv_ref, aux_ref):
    x = hs0_ref[...]
    logits = jax.lax.dot_general(
        x, wg_ref[...], (((1,), (0,)), ((), ())),
        preferred_element_type=jnp.float32)
    m = jnp.max(logits, axis=1, keepdims=True)
    ex = jnp.exp(logits - m)
    p = ex / jnp.sum(ex, axis=1, keepdims=True)

    iota = jax.lax.broadcasted_iota(jnp.int32, (B, E), 1)
    v1 = jnp.max(p, axis=1, keepdims=True)
    i1 = jnp.min(jnp.where(p == v1, iota, E), axis=1, keepdims=True)
    mask1 = iota == i1
    p2 = jnp.where(mask1, -1.0, p)
    v2 = jnp.max(p2, axis=1, keepdims=True)
    i2 = jnp.min(jnp.where(p2 == v2, iota, E), axis=1, keepdims=True)
    mask2 = iota == i2

    denom = jnp.maximum(v1 + v2, 1e-08)
    wv_ref[...] = jnp.concatenate([v1 / denom, v2 / denom], axis=1)
    topi_ref[...] = jnp.concatenate([i1, i2], axis=1)

    importance = jnp.sum(p, axis=0, keepdims=True) / B
    sel = (mask1 | mask2).astype(jnp.float32)
    load = jnp.sum(sel, axis=0, keepdims=True) / (B * K)
    aux_ref[...] = jnp.sum(E * importance * load, axis=1, keepdims=True)


def _route(hs0, wg):
    return pl.pallas_call(
        _router_kernel,
        in_specs=[pl.BlockSpec(memory_space=pltpu.VMEM),
                  pl.BlockSpec(memory_space=pltpu.VMEM)],
        out_specs=[pl.BlockSpec(memory_space=pltpu.VMEM)] * 3,
        out_shape=[
            jax.ShapeDtypeStruct((B, K), jnp.int32),
            jax.ShapeDtypeStruct((B, K), jnp.float32),
            jax.ShapeDtypeStruct((1, 1), jnp.float32),
        ],
    )(hs0, wg)


def _moe_kernel(topi_ref, wv_ref,
                hs_ref, cos_ref, sin_ref,
                wd_ref, wqkv_ref, wo_ref, wgu_ref, wdn_ref, wu_ref,
                out_ref,
                h_s, q_s, k_s, v_s):
    b = pl.program_id(0)
    kk = pl.program_id(1)
    wgt = wv_ref[b, kk]

    bf = jnp.bfloat16
    wd = wd_ref[0].astype(bf)
    wqkv = wqkv_ref[0].astype(bf)
    scale = np.float32(1.0 / np.sqrt(HD))

    def rope_head(xh, cosc, sinc):
        x1 = xh[:, :HD // 2]
        x2 = xh[:, HD // 2:]
        rot = jnp.concatenate([-x2, x1], axis=1)
        return xh * cosc + rot * sinc

    def phase1(c, carry):
        rows = pl.ds(c * CHUNK, CHUNK)
        x = hs_ref[0, rows, :].astype(bf)
        hc = jnp.dot(x, wd, preferred_element_type=jnp.float32)
        h_s[rows, :] = hc
        qkv = jnp.dot(hc.astype(bf), wqkv,
                      preferred_element_type=jnp.float32)
        cosc = cos_ref[rows, :]
        sinc = sin_ref[rows, :]
        for hh in range(NHS):
            qh = qkv[:, hh * HD:(hh + 1) * HD]
            kh = qkv[:, HS + hh * HD:HS + (hh + 1) * HD]
            vh = qkv[:, 2 * HS + hh * HD:2 * HS + (hh + 1) * HD]
            q_s[hh, rows, :] = (rope_head(qh, cosc, sinc) * scale).astype(bf)
            k_s[hh, rows, :] = rope_head(kh, cosc, sinc).astype(bf)
            v_s[hh, rows, :] = vh.astype(bf)
        return carry

    jax.lax.fori_loop(0, NCHUNK, phase1, 0)

    wo = wo_ref[0].astype(bf)
    wgu = wgu_ref[0].astype(bf)
    wdn = wdn_ref[0].astype(bf)
    wu = wu_ref[0].astype(bf)

    def phase2(c, carry):
        rows = pl.ds(c * CHUNK, CHUNK)
        hc = h_s[rows, :]
        o_heads = []
        for hh in range(NHS):
            qh = q_s[hh, rows, :]
            kh = k_s[hh]
            vh = v_s[hh]
            scores = jax.lax.dot_general(
                qh, kh, (((1,), (1,)), ((), ())),
                preferred_element_type=jnp.float32)
            mx = jnp.max(scores, axis=1, keepdims=True)
            ee = jnp.exp(scores - mx)
            rs = 1.0 / jnp.sum(ee, axis=1, keepdims=True)
            ov = jnp.dot(ee.astype(bf), vh,
                         preferred_element_type=jnp.float32)
            o_heads.append(ov * rs)
        o = jnp.concatenate(o_heads, axis=1).astype(bf)
        o = jnp.dot(o, wo, preferred_element_type=jnp.float32)
        t = _rms(hc + o)
        gu = jnp.dot(t.astype(bf), wgu,
                     preferred_element_type=jnp.float32)
        g = gu[:, :INTER]
        u = gu[:, INTER:]
        mm = jnp.dot((jax.nn.silu(g) * u).astype(bf), wdn,
                     preferred_element_type=jnp.float32)
        t2 = _rms(t + mm)
        oe = jnp.dot(t2.astype(bf), wu, preferred_element_type=jnp.float32)

        @pl.when(kk == 0)
        def _():
            out_ref[0, rows, :] = wgt * oe

        @pl.when(kk == K - 1)
        def _():
            x = hs_ref[0, rows, :] + out_ref[0, rows, :] + wgt * oe
            var = jnp.mean(x * x, axis=-1, keepdims=True)
            out_ref[0, rows, :] = x * jax.lax.rsqrt(var + EPS)

        return carry

    jax.lax.fori_loop(0, NCHUNK, phase2, 0)


def _moe(topi, wv, hs, cos, sin, Wd, Wqkv, Wo, Wgu, Wdn, Wu):
    grid_spec = pltpu.PrefetchScalarGridSpec(
        num_scalar_prefetch=2,
        grid=(B, K),
        in_specs=[
            pl.BlockSpec((1, S, H), lambda b, k, ti, wv_: (b, 0, 0)),
            pl.BlockSpec((S, HD), lambda b, k, ti, wv_: (0, 0)),
            pl.BlockSpec((S, HD), lambda b, k, ti, wv_: (0, 0)),
            pl.BlockSpec((1, H, HS), lambda b, k, ti, wv_: (ti[b, k], 0, 0)),
            pl.BlockSpec((1, HS, 3 * HS), lambda b, k, ti, wv_: (ti[b, k], 0, 0)),
            pl.BlockSpec((1, HS, HS), lambda b, k, ti, wv_: (ti[b, k], 0, 0)),
            pl.BlockSpec((1, HS, 2 * INTER), lambda b, k, ti, wv_: (ti[b, k], 0, 0)),
            pl.BlockSpec((1, INTER, HS), lambda b, k, ti, wv_: (ti[b, k], 0, 0)),
            pl.BlockSpec((1, HS, H), lambda b, k, ti, wv_: (ti[b, k], 0, 0)),
        ],
        out_specs=pl.BlockSpec((1, S, H), lambda b, k, ti, wv_: (b, 0, 0)),
        scratch_shapes=[
            pltpu.VMEM((S, HS), jnp.float32),
            pltpu.VMEM((NHS, S, HD), jnp.bfloat16),
            pltpu.VMEM((NHS, S, HD), jnp.bfloat16),
            pltpu.VMEM((NHS, S, HD), jnp.bfloat16),
        ],
    )
    return pl.pallas_call(
        _moe_kernel,
        grid_spec=grid_spec,
        out_shape=jax.ShapeDtypeStruct((B, S, H), jnp.float32),
        compiler_params=pltpu.CompilerParams(
            vmem_limit_bytes=64 * 1024 * 1024,
            dimension_semantics=("parallel", "arbitrary"),
        ),
    )(topi, wv, hs, cos, sin, Wd, Wqkv, Wo, Wgu, Wdn, Wu)


def _final_kernel(hs_ref, mix_ref, o_ref):
    x = hs_ref[...] + mix_ref[...]
    var = jnp.mean(x * x, axis=-1, keepdims=True)
    o_ref[...] = x * jax.lax.rsqrt(var + EPS)


def _finalize(hs, mixed):
    blk = pl.BlockSpec((1, 512, H), lambda b, c: (b, c, 0))
    return pl.pallas_call(
        _final_kernel,
        grid=(B, S // 512),
        in_specs=[blk, blk],
        out_specs=blk,
        out_shape=jax.ShapeDtypeStruct((B, S, H), jnp.float32),
    )(hs, mixed)


def kernel(hidden_states, input_injection, cos, sin, Wg, Wd, Wu, Wqkv, Wo, Wgu, Wdn):
    hs = _make_hs(hidden_states, input_injection)
    topi, wv, aux = _route(hs[:, 0, :], Wg)
    out = _moe(topi, wv, hs, cos, sin, Wd, Wqkv, Wo, Wgu, Wdn, Wu)
    return out, aux.reshape(())
